# baseline (speedup 1.0000x reference)
import functools

import jax
import jax.numpy as jnp
from jax import lax
from jax.experimental import pallas as pl
from jax.experimental.pallas import tpu as pltpu
from jax.experimental.pallas import tpu_sc as plsc

_INFO = plsc.get_sparse_core_info()
_NC = _INFO.num_cores
_NS = _INFO.num_subcores
_NW = _NC * _NS
_L = 16

_CHUNK = 128
_DPAD = 128
_NBUF = 4


def _transpose_block(src_tiles, dst_rows, n_rows):
    iota = lax.iota(jnp.int32, _L)

    def per_v(v, carry):
        vvec = jnp.full((_L,), v, dtype=jnp.int32)
        for g in range(_DPAD // _L):
            dvec = iota + (g * _L)
            it = lax.shift_right_logical(dvec, 3)
            ir = lax.bitwise_and(dvec, 7)
            vec = plsc.load_gather(src_tiles, [it, ir, vvec])
            dst_rows[v, pl.ds(g * _L, _L)] = vec
        return carry

    lax.fori_loop(0, n_rows, per_v, 0)


def _make_transpose(v_total: int, d: int):
    mesh = plsc.VectorSubcoreMesh(core_axis_name="c", subcore_axis_name="s")
    n_full = v_total // _CHUNK
    v_rem = v_total - n_full * _CHUNK
    blocks_per_w = (n_full + _NW - 1) // _NW
    n_dtiles = (d + 7) // 8

    @functools.partial(
        pl.kernel,
        mesh=mesh,
        compiler_params=pltpu.CompilerParams(needs_layout_passes=False),
        out_type=jax.ShapeDtypeStruct((v_total, _DPAD), jnp.float32),
        scratch_types=[
            pltpu.VMEM((16, 8, _CHUNK), jnp.float32),
            pltpu.VMEM((_CHUNK, _DPAD), jnp.float32),
            pltpu.VMEM((16, 8, v_rem), jnp.float32),
            pltpu.VMEM((v_rem, _DPAD), jnp.float32),
            pltpu.SemaphoreType.DMA,
        ],
    )
    def transpose_kernel(src_hbm, tpad_hbm, stage, rows, stage2, rows2, tsem):
        wid = lax.axis_index("s") * _NC + lax.axis_index("c")

        def block(k, carry):
            blk = wid + k * _NW

            @pl.when(blk < n_full)
            def _do():
                v0 = blk * _CHUNK
                hs = []
                for t in range(n_dtiles):
                    r0, nr = 8 * t, min(8, d - 8 * t)
                    hs.append(pltpu.async_copy(
                        src_hbm.at[pl.ds(r0, nr), pl.ds(v0, _CHUNK)],
                        stage.at[t, pl.ds(0, nr)],
                        tsem,
                    ))
                for h in hs:
                    h.wait()
                _transpose_block(stage, rows, _CHUNK)
                pltpu.sync_copy(rows, tpad_hbm.at[pl.ds(v0, _CHUNK)])

            return carry

        lax.fori_loop(0, blocks_per_w, block, 0)

        @pl.when(wid == _NW - 1)
        def _seam():
            v0 = n_full * _CHUNK
            hs = []
            for t in range(n_dtiles):
                r0, nr = 8 * t, min(8, d - 8 * t)
                hs.append(pltpu.async_copy(
                    src_hbm.at[pl.ds(r0, nr), pl.ds(v0, v_rem)],
                    stage2.at[t, pl.ds(0, nr)],
                    tsem,
                ))
            for h in hs:
                h.wait()
            _transpose_block(stage2, rows2, v_rem)
            pltpu.sync_copy(rows2, tpad_hbm.at[pl.ds(v0, v_rem)])

    return transpose_kernel


def _make_gather(n_chunks: int):
    mesh = plsc.VectorSubcoreMesh(core_axis_name="c", subcore_axis_name="s")
    b_per_w = n_chunks * _CHUNK
    total = _NW * b_per_w
    n_grp = n_chunks // _NBUF

    @functools.partial(
        pl.kernel,
        mesh=mesh,
        out_type=jax.ShapeDtypeStruct((total, _DPAD), jnp.float32),
        scratch_types=[
            pltpu.VMEM((n_chunks, _CHUNK), jnp.int32),
            [pltpu.VMEM((_CHUNK, _DPAD), jnp.float32) for _ in range(_NBUF)],
            [pltpu.SemaphoreType.DMA for _ in range(_NBUF)],
            [pltpu.SemaphoreType.DMA for _ in range(_NBUF)],
        ],
    )
    def gather_kernel(idx_hbm, table_hbm, out_hbm, idx_v, rows, gsems, ssems):
        wid = lax.axis_index("s") * _NC + lax.axis_index("c")
        pltpu.sync_copy(idx_hbm.at[wid], idx_v)
        base = wid * b_per_w

        def body(k, carry):
            c0 = k * _NBUF

            @pl.when(k > 0)
            def _drain_prev():
                for i in range(_NBUF):
                    pltpu.make_async_copy(
                        rows[i], out_hbm.at[pl.ds(base, _CHUNK)], ssems[i]
                    ).wait()

            gathers = [
                pltpu.async_copy(
                    table_hbm.at[idx_v.at[c0 + i]], rows[i], gsems[i]
                )
                for i in range(_NBUF)
            ]
            for i in range(_NBUF):
                gathers[i].wait()
                pltpu.async_copy(
                    rows[i],
                    out_hbm.at[pl.ds(base + (c0 + i) * _CHUNK, _CHUNK)],
                    ssems[i],
                )
            return carry

        lax.fori_loop(0, n_grp, body, 0)
        for i in range(_NBUF):
            pltpu.make_async_copy(
                rows[i], out_hbm.at[pl.ds(base, _CHUNK)], ssems[i]
            ).wait()

    return gather_kernel


def kernel(x, table):
    b = x.shape[0] * x.shape[1]
    v, d = table.shape
    n_chunks = b // (_NW * _CHUNK)
    idx = jnp.reshape(x.astype(jnp.int32), (_NW, n_chunks, _CHUNK))
    tpad = _make_transpose(v, d)(jnp.swapaxes(table, 0, 1))
    out = _make_gather(n_chunks)(idx, tpad)
    return jnp.reshape(out[:, :d], (x.shape[0], x.shape[1], d))


# speedup vs baseline: 1.4674x; 1.4674x over previous
import functools

import jax
import jax.numpy as jnp
from jax import lax
from jax.experimental import pallas as pl
from jax.experimental.pallas import tpu as pltpu
from jax.experimental.pallas import tpu_sc as plsc

_INFO = plsc.get_sparse_core_info()
_NC = _INFO.num_cores
_NS = _INFO.num_subcores
_NW = _NC * _NS
_L = 16

_CHUNK = 128
_DPAD = 128
_NBUF = 4


def _transpose_block(src_rows, dst_rows, n_rows):
    iota = lax.iota(jnp.int32, _L)
    dvecs = [iota + (g * _L) for g in range(_DPAD // _L)]

    @functools.partial(plsc.parallel_loop, 0, n_rows)
    def per_v(v):
        vvec = jnp.full((_L,), v, dtype=jnp.int32)
        for g in range(_DPAD // _L):
            vec = plsc.load_gather(src_rows, [dvecs[g], vvec])
            dst_rows[v, pl.ds(g * _L, _L)] = vec


def _make_transpose(v_total: int, d: int):
    mesh = plsc.VectorSubcoreMesh(core_axis_name="c", subcore_axis_name="s")
    n_full = v_total // _CHUNK
    v_rem = v_total - n_full * _CHUNK
    blocks_per_w = (n_full + _NW - 1) // _NW
    n_dtiles = (d + 7) // 8

    @functools.partial(
        pl.kernel,
        mesh=mesh,
        compiler_params=pltpu.CompilerParams(needs_layout_passes=False),
        out_type=jax.ShapeDtypeStruct((v_total, _DPAD), jnp.float32),
        scratch_types=[
            pltpu.VMEM((_DPAD, _CHUNK), jnp.float32),
            pltpu.VMEM((_CHUNK, _DPAD), jnp.float32),
            pltpu.VMEM((_DPAD, v_rem), jnp.float32),
            pltpu.VMEM((v_rem, _DPAD), jnp.float32),
            pltpu.SemaphoreType.DMA,
        ],
    )
    def transpose_kernel(src_hbm, tpad_hbm, stage, rows, stage2, rows2, tsem):
        wid = lax.axis_index("s") * _NC + lax.axis_index("c")

        def block(k, carry):
            blk = wid + k * _NW

            @pl.when(blk < n_full)
            def _do():
                v0 = blk * _CHUNK
                hs = []
                for t in range(n_dtiles):
                    r0, nr = 8 * t, min(8, d - 8 * t)
                    hs.append(pltpu.async_copy(
                        src_hbm.at[pl.ds(r0, nr), pl.ds(v0, _CHUNK)],
                        stage.at[pl.ds(r0, nr)],
                        tsem,
                    ))
                for h in hs:
                    h.wait()
                _transpose_block(stage, rows, _CHUNK)
                pltpu.sync_copy(rows, tpad_hbm.at[pl.ds(v0, _CHUNK)])

            return carry

        lax.fori_loop(0, blocks_per_w, block, 0)

        @pl.when(wid == _NW - 1)
        def _seam():
            v0 = n_full * _CHUNK
            hs = []
            for t in range(n_dtiles):
                r0, nr = 8 * t, min(8, d - 8 * t)
                hs.append(pltpu.async_copy(
                    src_hbm.at[pl.ds(r0, nr), pl.ds(v0, v_rem)],
                    stage2.at[pl.ds(r0, nr)],
                    tsem,
                ))
            for h in hs:
                h.wait()
            _transpose_block(stage2, rows2, v_rem)
            pltpu.sync_copy(rows2, tpad_hbm.at[pl.ds(v0, v_rem)])

    return transpose_kernel


def _make_gather(n_chunks: int):
    mesh = plsc.VectorSubcoreMesh(core_axis_name="c", subcore_axis_name="s")
    b_per_w = n_chunks * _CHUNK
    total = _NW * b_per_w
    n_grp = n_chunks // _NBUF

    @functools.partial(
        pl.kernel,
        mesh=mesh,
        out_type=jax.ShapeDtypeStruct((total, _DPAD), jnp.float32),
        scratch_types=[
            pltpu.VMEM((n_chunks, _CHUNK), jnp.int32),
            [pltpu.VMEM((_CHUNK, _DPAD), jnp.float32) for _ in range(_NBUF)],
            [pltpu.SemaphoreType.DMA for _ in range(_NBUF)],
            [pltpu.SemaphoreType.DMA for _ in range(_NBUF)],
        ],
    )
    def gather_kernel(idx_hbm, table_hbm, out_hbm, idx_v, rows, gsems, ssems):
        wid = lax.axis_index("s") * _NC + lax.axis_index("c")
        pltpu.sync_copy(idx_hbm.at[wid], idx_v)
        base = wid * b_per_w

        def body(k, carry):
            c0 = k * _NBUF

            @pl.when(k > 0)
            def _drain_prev():
                for i in range(_NBUF):
                    pltpu.make_async_copy(
                        rows[i], out_hbm.at[pl.ds(base, _CHUNK)], ssems[i]
                    ).wait()

            gathers = [
                pltpu.async_copy(
                    table_hbm.at[idx_v.at[c0 + i]], rows[i], gsems[i]
                )
                for i in range(_NBUF)
            ]
            for i in range(_NBUF):
                gathers[i].wait()
                pltpu.async_copy(
                    rows[i],
                    out_hbm.at[pl.ds(base + (c0 + i) * _CHUNK, _CHUNK)],
                    ssems[i],
                )
            return carry

        lax.fori_loop(0, n_grp, body, 0)
        for i in range(_NBUF):
            pltpu.make_async_copy(
                rows[i], out_hbm.at[pl.ds(base, _CHUNK)], ssems[i]
            ).wait()

    return gather_kernel


def kernel(x, table):
    b = x.shape[0] * x.shape[1]
    v, d = table.shape
    n_chunks = b // (_NW * _CHUNK)
    idx = jnp.reshape(x.astype(jnp.int32), (_NW, n_chunks, _CHUNK))
    tpad = _make_transpose(v, d)(jnp.swapaxes(table, 0, 1))
    out = _make_gather(n_chunks)(idx, tpad)
    return jnp.reshape(out[:, :d], (x.shape[0], x.shape[1], d))
